# Initial kernel scaffold; baseline (speedup 1.0000x reference)
import jax
import jax.numpy as jnp
from jax.experimental import pallas as pl


def kernel(x, pos, edge_index, batch, W0, W5, W6, W7, Wfc):
    raise NotImplementedError("write your pallas kernel here")


# speedup vs baseline: 10.7876x; 10.7876x over previous
import functools

import jax
import jax.numpy as jnp
from jax import lax
from jax.experimental import pallas as pl
from jax.experimental.pallas import tpu as pltpu
import jax.experimental.pallas.tpu_sc as plsc

N = 100000
E = 1600000
NB = 16
GX, GY = 16, 12
NCELL = GX * GY

NC, NS = 2, 16
NW = NC * NS
N_PAD = 100352
E_ROWS = E // 128
E_ROWS_PAD = 12544
RW_INIT = N_PAD // NS
SEG_ROWS = 3264

_f32 = jnp.float32
_i32 = jnp.int32

_SC_MESH = dict(core_axis_name="c", subcore_axis_name="s",
                num_cores=NC, num_subcores=NS)


def _elu(t):
    return jnp.where(t > 0, t, jnp.exp(t) - 1.0)


def _make_edge_pass(w, edge_split):
    rows_w = E_ROWS_PAD // (NW if edge_split else NS)
    GJ = 8
    G, T = divmod(rows_w, GJ)
    mesh = plsc.VectorSubcoreMesh(**_SC_MESH)

    def body(tabA, tabB, src2d, dst2d, out0, out1, acc, sbuf, dbuf, vbuf,
             gsem, ssem):
        c = lax.axis_index("c")
        sid = lax.axis_index("s")
        wid = sid * NC + c
        r0 = sid * RW_INIT

        def run(tab, out):
            pltpu.sync_copy(tab.at[pl.ds(r0, RW_INIT)],
                            acc.at[pl.ds(r0, RW_INIT)])
            plsc.subcore_barrier()
            base0 = (wid if edge_split else sid) * rows_w

            def group(base, nrows):
                pltpu.sync_copy(src2d.at[pl.ds(base, nrows)],
                                sbuf.at[pl.ds(0, nrows)])
                pltpu.sync_copy(dst2d.at[pl.ds(base, nrows)],
                                dbuf.at[pl.ds(0, nrows)])
                ds = [pltpu.async_copy(tab.at[sbuf.at[j]], vbuf.at[j], gsem)
                      for j in range(nrows)]
                for d in ds:
                    d.wait()
                ds = [pltpu.async_copy(vbuf.at[j], acc.at[dbuf.at[j]], ssem,
                                       add=True)
                      for j in range(nrows)]
                for d in ds:
                    d.wait()

            def lbody(g, carry):
                group(base0 + g * GJ, GJ)
                return carry

            lax.fori_loop(0, G, lbody, 0)
            if T:
                group(base0 + G * GJ, T)
            plsc.subcore_barrier()
            pltpu.sync_copy(acc.at[pl.ds(r0, RW_INIT)],
                            out.at[pl.ds(r0, RW_INIT)])

        @pl.when(c == 0)
        def _():
            run(tabA, out0)

        @pl.when(c == 1)
        def _():
            run(tabB, out1)

    sds = jax.ShapeDtypeStruct((N_PAD, w), _f32)
    return pl.kernel(
        body,
        out_type=[sds, sds],
        mesh=mesh,
        compiler_params=pltpu.CompilerParams(use_tc_tiling_on_sc=False),
        scratch_types=[
            pltpu.VMEM_SHARED((N_PAD, w), _f32),
            pltpu.VMEM((GJ, 128), _i32),
            pltpu.VMEM((GJ, 128), _i32),
            pltpu.VMEM((GJ, 128, w), _f32),
            pltpu.SemaphoreType.DMA,
            pltpu.SemaphoreType.DMA,
        ],
    )


_edge_pass_w16e = _make_edge_pass(16, True)
_edge_pass_w16f = _make_edge_pass(16, False)

_POOL_CH = 224
_POOL_NCH = 3136 // _POOL_CH


def _pool_body(hA, hB, sg, out, acc, bufA, bufB, sbuf):
    c = lax.axis_index("c")
    sid = lax.axis_index("s")
    wid = sid * NC + c
    base = wid * 3136
    neg = jnp.full((16,), -jnp.inf, _f32)

    def ib(r, carry):
        acc[r, pl.ds(0, 16)] = neg
        acc[r, pl.ds(16, 16)] = neg
        return carry

    lax.fori_loop(0, SEG_ROWS, ib, 0)

    def cb(k, carry):
        b = base + k * _POOL_CH
        pltpu.sync_copy(hA.at[pl.ds(b, _POOL_CH)], bufA)
        pltpu.sync_copy(hB.at[pl.ds(b, _POOL_CH)], bufB)
        pltpu.sync_copy(sg.at[pl.ds(b, _POOL_CH)], sbuf)

        def nb(g, carry2):
            segv = sbuf[pl.ds(g * 16, 16)]
            for j in range(16):
                s = segv[j]
                i = g * 16 + j
                acc[s, pl.ds(0, 16)] = jnp.maximum(acc[s, pl.ds(0, 16)],
                                                   bufA[i])
                acc[s, pl.ds(16, 16)] = jnp.maximum(acc[s, pl.ds(16, 16)],
                                                    bufB[i])
            return carry2

        lax.fori_loop(0, _POOL_CH // 16, nb, 0)
        return carry

    lax.fori_loop(0, _POOL_NCH, cb, 0)
    pltpu.sync_copy(acc, out.at[wid])


_pool = pl.kernel(
    _pool_body,
    out_type=jax.ShapeDtypeStruct((NW, SEG_ROWS, 32), _f32),
    mesh=plsc.VectorSubcoreMesh(**_SC_MESH),
    compiler_params=pltpu.CompilerParams(use_tc_tiling_on_sc=False),
    scratch_types=[
        pltpu.VMEM((SEG_ROWS, 32), _f32),
        pltpu.VMEM((_POOL_CH, 16), _f32),
        pltpu.VMEM((_POOL_CH, 16), _f32),
        pltpu.VMEM((_POOL_CH,), _i32),
    ],
)


_BR = 1024
_NBLK = N_PAD // _BR


def _tcpre(xp, W0):
    def body(x_r, w_r, o_r):
        o_r[...] = x_r[...] * w_r[...]

    return pl.pallas_call(
        body,
        grid=(_NBLK,),
        in_specs=[pl.BlockSpec((_BR, 1), lambda i: (i, 0)),
                  pl.BlockSpec((1, 16), lambda i: (0, 0))],
        out_specs=pl.BlockSpec((_BR, 16), lambda i: (i, 0)),
        out_shape=jax.ShapeDtypeStruct((N_PAD, 16), _f32),
    )(xp, W0)


def _tc0(p0, p1, t0):
    def body(a_r, b_r, t_r, o_r):
        o_r[...] = _elu(a_r[...] + b_r[...] - t_r[...])

    blk = pl.BlockSpec((_BR, 16), lambda i: (i, 0))
    return pl.pallas_call(
        body,
        grid=(_NBLK,),
        in_specs=[blk, blk, blk],
        out_specs=blk,
        out_shape=jax.ShapeDtypeStruct((N_PAD, 16), _f32),
    )(p0, p1, t0)


def _tc5(p0, p1, h1, W5):
    def body(a_r, b_r, h_r, w_r, oA, oB):
        t = a_r[...] + b_r[...] - h_r[...]
        h2 = _elu(jnp.dot(t, w_r[...], preferred_element_type=_f32))
        oA[...] = h2[:, :16]
        oB[...] = h2[:, 16:]

    blk = pl.BlockSpec((_BR, 16), lambda i: (i, 0))
    sds = jax.ShapeDtypeStruct((N_PAD, 16), _f32)
    return pl.pallas_call(
        body,
        grid=(_NBLK,),
        in_specs=[blk, blk, blk, pl.BlockSpec((16, 32), lambda i: (0, 0))],
        out_specs=[blk, blk],
        out_shape=[sds, sds],
    )(p0, p1, h1, W5)


def _tc6(pA, pB, W6):
    def body(a_r, b_r, w_r, oA, oB):
        w = w_r[...]
        t = (jnp.dot(a_r[...], w[:16, :], preferred_element_type=_f32)
             + jnp.dot(b_r[...], w[16:, :], preferred_element_type=_f32))
        h3 = _elu(t)
        oA[...] = h3[:, :16]
        oB[...] = h3[:, 16:]

    blk = pl.BlockSpec((_BR, 16), lambda i: (i, 0))
    sds = jax.ShapeDtypeStruct((N_PAD, 16), _f32)
    return pl.pallas_call(
        body,
        grid=(_NBLK,),
        in_specs=[blk, blk, pl.BlockSpec((32, 32), lambda i: (0, 0))],
        out_specs=[blk, blk],
        out_shape=[sds, sds],
    )(pA, pB, W6)


def _tc7(pA, pB, W7, hA, hB, pxp, pyp, btp):
    def body(a_r, b_r, w_r, hA_r, hB_r, px_r, py_r, bt_r, o4A, o4B, oseg):
        w = w_r[...]
        t = (jnp.dot(a_r[...], w[:16, :], preferred_element_type=_f32)
             + jnp.dot(b_r[...], w[16:, :], preferred_element_type=_f32))
        u = _elu(t)
        o4A[...] = u[:, :16] + hA_r[...]
        o4B[...] = u[:, 16:] + hB_r[...]
        cx = jnp.clip(jnp.floor(px_r[...] * GX).astype(_i32), 0, GX - 1)
        cy = jnp.clip(jnp.floor(py_r[...] * GY).astype(_i32), 0, GY - 1)
        oseg[...] = bt_r[...] * NCELL + cx * GY + cy

    blk = pl.BlockSpec((_BR, 16), lambda i: (i, 0))
    col = pl.BlockSpec((_BR, 1), lambda i: (i, 0))
    sds16 = jax.ShapeDtypeStruct((N_PAD, 16), _f32)
    return pl.pallas_call(
        body,
        grid=(_NBLK,),
        in_specs=[blk, blk, pl.BlockSpec((32, 32), lambda i: (0, 0)),
                  blk, blk, col, col, col],
        out_specs=[blk, blk, col],
        out_shape=[sds16, sds16, jax.ShapeDtypeStruct((N_PAD, 1), _i32)],
    )(pA, pB, W7, hA, hB, pxp, pyp, btp)


def _tcred(parts):
    def body(p_r, o_r):
        m = p_r[0]
        for i in range(1, NW):
            m = jnp.maximum(m, p_r[i])
        o_r[...] = jnp.where(jnp.isfinite(m), m, 0.0)

    return pl.pallas_call(
        body,
        grid=(24,),
        in_specs=[pl.BlockSpec((NW, 128, 32), lambda i: (0, i, 0))],
        out_specs=pl.BlockSpec((128, 32), lambda i: (i, 0)),
        out_shape=jax.ShapeDtypeStruct((3072, 32), _f32),
    )(parts)


def _tcfc(feat, Wfc):
    def body(f_r, w_r, o_r):
        o_r[...] = jnp.dot(f_r[...], w_r[...], preferred_element_type=_f32)

    return pl.pallas_call(
        body,
        out_shape=jax.ShapeDtypeStruct((NB, 2), _f32),
    )(feat, Wfc)


def kernel(x, pos, edge_index, batch, W0, W5, W6, W7, Wfc):
    src = edge_index[0].astype(_i32)
    dst = edge_index[1].astype(_i32)
    ep = E_ROWS_PAD * 128 - E
    pad_ar = jnp.arange(ep, dtype=_i32)
    pad_idx = N + pad_ar % (N_PAD - N)
    src2d = jnp.concatenate([src, pad_idx]).reshape(E_ROWS_PAD, 128)
    dst2d = jnp.concatenate([dst, pad_idx]).reshape(E_ROWS_PAD, 128)

    npad = N_PAD - N
    xp = jnp.pad(x.astype(_f32), ((0, npad), (0, 0)))
    t0 = _tcpre(xp, W0)
    P0, P1 = _edge_pass_w16e(t0, t0, src2d, dst2d)
    h1 = _tc0(P0, P1, t0)
    P0, P1 = _edge_pass_w16e(h1, h1, src2d, dst2d)
    h2A, h2B = _tc5(P0, P1, h1, W5)
    PA, PB = _edge_pass_w16f(h2A, h2B, src2d, dst2d)
    h3A, h3B = _tc6(PA, PB, W6)
    PA, PB = _edge_pass_w16f(h3A, h3B, src2d, dst2d)

    pxp = jnp.pad(pos[:, 0:1], ((0, npad), (0, 0)))
    pyp = jnp.pad(pos[:, 1:2], ((0, npad), (0, 0)))
    btp = jnp.pad(batch.astype(_i32), (0, npad),
                  constant_values=NB).reshape(N_PAD, 1)
    h4A, h4B, segp = _tc7(PA, PB, W7, h2A, h2B, pxp, pyp, btp)

    parts = _pool(h4A, h4B, segp.reshape(N_PAD))
    pooled = _tcred(parts)
    feat = pooled.reshape(NB, NCELL * 32)
    return _tcfc(feat, Wfc)


# speedup vs baseline: 11.5928x; 1.0746x over previous
import functools

import jax
import jax.numpy as jnp
from jax import lax
from jax.experimental import pallas as pl
from jax.experimental.pallas import tpu as pltpu
import jax.experimental.pallas.tpu_sc as plsc

N = 100000
E = 1600000
NB = 16
GX, GY = 16, 12
NCELL = GX * GY

NC, NS = 2, 16
NW = NC * NS
N_PAD = 100352
E_ROWS = E // 128
E_ROWS_PAD = 12544
RW_INIT = N_PAD // NS
SEG_ROWS = 3264

_f32 = jnp.float32
_i32 = jnp.int32

_SC_MESH = dict(core_axis_name="c", subcore_axis_name="s",
                num_cores=NC, num_subcores=NS)


def _elu(t):
    return jnp.where(t > 0, t, jnp.exp(t) - 1.0)


def _make_edge_pass(w, edge_split):
    rows_w = E_ROWS_PAD // (NW if edge_split else NS)
    GJ = 4
    NG = rows_w // GJ
    H = NG // 2
    assert NG % 2 == 0 and NG * GJ == rows_w
    mesh = plsc.VectorSubcoreMesh(**_SC_MESH)

    def body(tabA, tabB, src2d, dst2d, out0, out1, acc,
             sb0, db0, vb0, sb1, db1, vb1, gsem0, gsem1, ssem0, ssem1):
        c = lax.axis_index("c")
        sid = lax.axis_index("s")
        wid = sid * NC + c
        r0 = sid * RW_INIT

        def run(tab, out):
            pltpu.sync_copy(tab.at[pl.ds(r0, RW_INIT)],
                            acc.at[pl.ds(r0, RW_INIT)])
            plsc.subcore_barrier()
            base0 = (wid if edge_split else sid) * rows_w

            def load_idx(sb, db, grp):
                base = base0 + grp * GJ
                pltpu.sync_copy(src2d.at[pl.ds(base, GJ)], sb)
                pltpu.sync_copy(dst2d.at[pl.ds(base, GJ)], db)

            def fire_gathers(sb, vb, sem):
                for j in range(GJ):
                    pltpu.async_copy(tab.at[sb.at[j]], vb.at[j], sem)

            def wait_gathers(sb, vb, sem):
                for j in range(GJ):
                    pltpu.make_async_copy(tab.at[sb.at[j]], vb.at[j],
                                          sem).wait()

            def scatters(db, vb, sem):
                ds = [pltpu.async_copy(vb.at[j], acc.at[db.at[j]], sem,
                                       add=True) for j in range(GJ)]
                for d in ds:
                    d.wait()

            load_idx(sb0, db0, 0)
            fire_gathers(sb0, vb0, gsem0)

            def pair(t, carry):
                g = 2 * t
                load_idx(sb1, db1, g + 1)
                fire_gathers(sb1, vb1, gsem1)
                wait_gathers(sb0, vb0, gsem0)
                scatters(db0, vb0, ssem0)
                @pl.when(t + 1 < H)
                def _():
                    load_idx(sb0, db0, g + 2)
                    fire_gathers(sb0, vb0, gsem0)
                wait_gathers(sb1, vb1, gsem1)
                scatters(db1, vb1, ssem1)
                return carry

            lax.fori_loop(0, H, pair, 0)
            plsc.subcore_barrier()
            pltpu.sync_copy(acc.at[pl.ds(r0, RW_INIT)],
                            out.at[pl.ds(r0, RW_INIT)])

        @pl.when(c == 0)
        def _():
            run(tabA, out0)

        @pl.when(c == 1)
        def _():
            run(tabB, out1)

    sds = jax.ShapeDtypeStruct((N_PAD, w), _f32)
    idx_t = pltpu.VMEM((GJ, 128), _i32)
    val_t = pltpu.VMEM((GJ, 128, w), _f32)
    return pl.kernel(
        body,
        out_type=[sds, sds],
        mesh=mesh,
        compiler_params=pltpu.CompilerParams(use_tc_tiling_on_sc=False),
        scratch_types=[
            pltpu.VMEM_SHARED((N_PAD, w), _f32),
            idx_t, idx_t, val_t,
            idx_t, idx_t, val_t,
            pltpu.SemaphoreType.DMA,
            pltpu.SemaphoreType.DMA,
            pltpu.SemaphoreType.DMA,
            pltpu.SemaphoreType.DMA,
        ],
    )


_edge_pass_w16e = _make_edge_pass(16, True)
_edge_pass_w16f = _make_edge_pass(16, False)

_POOL_CH = 224
_POOL_NCH = 3136 // _POOL_CH


def _pool_body(hA, hB, sg, out, acc, bufA, bufB, sbuf):
    c = lax.axis_index("c")
    sid = lax.axis_index("s")
    wid = sid * NC + c
    base = wid * 3136
    neg = jnp.full((16,), -jnp.inf, _f32)

    def ib(r, carry):
        acc[r, pl.ds(0, 16)] = neg
        acc[r, pl.ds(16, 16)] = neg
        return carry

    lax.fori_loop(0, SEG_ROWS, ib, 0)

    def cb(k, carry):
        b = base + k * _POOL_CH
        pltpu.sync_copy(hA.at[pl.ds(b, _POOL_CH)], bufA)
        pltpu.sync_copy(hB.at[pl.ds(b, _POOL_CH)], bufB)
        pltpu.sync_copy(sg.at[pl.ds(b, _POOL_CH)], sbuf)

        def nb(g, carry2):
            segv = sbuf[pl.ds(g * 16, 16)]
            for j in range(16):
                s = segv[j]
                i = g * 16 + j
                acc[s, pl.ds(0, 16)] = jnp.maximum(acc[s, pl.ds(0, 16)],
                                                   bufA[i])
                acc[s, pl.ds(16, 16)] = jnp.maximum(acc[s, pl.ds(16, 16)],
                                                    bufB[i])
            return carry2

        lax.fori_loop(0, _POOL_CH // 16, nb, 0)
        return carry

    lax.fori_loop(0, _POOL_NCH, cb, 0)
    pltpu.sync_copy(acc, out.at[wid])


_pool = pl.kernel(
    _pool_body,
    out_type=jax.ShapeDtypeStruct((NW, SEG_ROWS, 32), _f32),
    mesh=plsc.VectorSubcoreMesh(**_SC_MESH),
    compiler_params=pltpu.CompilerParams(use_tc_tiling_on_sc=False),
    scratch_types=[
        pltpu.VMEM((SEG_ROWS, 32), _f32),
        pltpu.VMEM((_POOL_CH, 16), _f32),
        pltpu.VMEM((_POOL_CH, 16), _f32),
        pltpu.VMEM((_POOL_CH,), _i32),
    ],
)


_BR = 1024
_NBLK = N_PAD // _BR


def _tcpre(xp, W0):
    def body(x_r, w_r, o_r):
        o_r[...] = x_r[...] * w_r[...]

    return pl.pallas_call(
        body,
        grid=(_NBLK,),
        in_specs=[pl.BlockSpec((_BR, 1), lambda i: (i, 0)),
                  pl.BlockSpec((1, 16), lambda i: (0, 0))],
        out_specs=pl.BlockSpec((_BR, 16), lambda i: (i, 0)),
        out_shape=jax.ShapeDtypeStruct((N_PAD, 16), _f32),
    )(xp, W0)


def _tc0(p0, p1, t0):
    def body(a_r, b_r, t_r, o_r):
        o_r[...] = _elu(a_r[...] + b_r[...] - t_r[...])

    blk = pl.BlockSpec((_BR, 16), lambda i: (i, 0))
    return pl.pallas_call(
        body,
        grid=(_NBLK,),
        in_specs=[blk, blk, blk],
        out_specs=blk,
        out_shape=jax.ShapeDtypeStruct((N_PAD, 16), _f32),
    )(p0, p1, t0)


def _tc5(p0, p1, h1, W5):
    def body(a_r, b_r, h_r, w_r, oA, oB):
        t = a_r[...] + b_r[...] - h_r[...]
        h2 = _elu(jnp.dot(t, w_r[...], preferred_element_type=_f32))
        oA[...] = h2[:, :16]
        oB[...] = h2[:, 16:]

    blk = pl.BlockSpec((_BR, 16), lambda i: (i, 0))
    sds = jax.ShapeDtypeStruct((N_PAD, 16), _f32)
    return pl.pallas_call(
        body,
        grid=(_NBLK,),
        in_specs=[blk, blk, blk, pl.BlockSpec((16, 32), lambda i: (0, 0))],
        out_specs=[blk, blk],
        out_shape=[sds, sds],
    )(p0, p1, h1, W5)


def _tc6(pA, pB, W6):
    def body(a_r, b_r, w_r, oA, oB):
        w = w_r[...]
        t = (jnp.dot(a_r[...], w[:16, :], preferred_element_type=_f32)
             + jnp.dot(b_r[...], w[16:, :], preferred_element_type=_f32))
        h3 = _elu(t)
        oA[...] = h3[:, :16]
        oB[...] = h3[:, 16:]

    blk = pl.BlockSpec((_BR, 16), lambda i: (i, 0))
    sds = jax.ShapeDtypeStruct((N_PAD, 16), _f32)
    return pl.pallas_call(
        body,
        grid=(_NBLK,),
        in_specs=[blk, blk, pl.BlockSpec((32, 32), lambda i: (0, 0))],
        out_specs=[blk, blk],
        out_shape=[sds, sds],
    )(pA, pB, W6)


def _tc7(pA, pB, W7, hA, hB, pxp, pyp, btp):
    def body(a_r, b_r, w_r, hA_r, hB_r, px_r, py_r, bt_r, o4A, o4B, oseg):
        w = w_r[...]
        t = (jnp.dot(a_r[...], w[:16, :], preferred_element_type=_f32)
             + jnp.dot(b_r[...], w[16:, :], preferred_element_type=_f32))
        u = _elu(t)
        o4A[...] = u[:, :16] + hA_r[...]
        o4B[...] = u[:, 16:] + hB_r[...]
        cx = jnp.clip(jnp.floor(px_r[...] * GX).astype(_i32), 0, GX - 1)
        cy = jnp.clip(jnp.floor(py_r[...] * GY).astype(_i32), 0, GY - 1)
        oseg[...] = bt_r[...] * NCELL + cx * GY + cy

    blk = pl.BlockSpec((_BR, 16), lambda i: (i, 0))
    col = pl.BlockSpec((_BR, 1), lambda i: (i, 0))
    sds16 = jax.ShapeDtypeStruct((N_PAD, 16), _f32)
    return pl.pallas_call(
        body,
        grid=(_NBLK,),
        in_specs=[blk, blk, pl.BlockSpec((32, 32), lambda i: (0, 0)),
                  blk, blk, col, col, col],
        out_specs=[blk, blk, col],
        out_shape=[sds16, sds16, jax.ShapeDtypeStruct((N_PAD, 1), _i32)],
    )(pA, pB, W7, hA, hB, pxp, pyp, btp)


def _tcred(parts):
    def body(p_r, o_r):
        m = p_r[0]
        for i in range(1, NW):
            m = jnp.maximum(m, p_r[i])
        o_r[...] = jnp.where(jnp.isfinite(m), m, 0.0)

    return pl.pallas_call(
        body,
        grid=(24,),
        in_specs=[pl.BlockSpec((NW, 128, 32), lambda i: (0, i, 0))],
        out_specs=pl.BlockSpec((128, 32), lambda i: (i, 0)),
        out_shape=jax.ShapeDtypeStruct((3072, 32), _f32),
    )(parts)


def _tcfc(feat, Wfc):
    def body(f_r, w_r, o_r):
        o_r[...] = jnp.dot(f_r[...], w_r[...], preferred_element_type=_f32)

    return pl.pallas_call(
        body,
        out_shape=jax.ShapeDtypeStruct((NB, 2), _f32),
    )(feat, Wfc)


def kernel(x, pos, edge_index, batch, W0, W5, W6, W7, Wfc):
    src = edge_index[0].astype(_i32)
    dst = edge_index[1].astype(_i32)
    ep = E_ROWS_PAD * 128 - E
    pad_ar = jnp.arange(ep, dtype=_i32)
    pad_idx = N + pad_ar % (N_PAD - N)
    src2d = jnp.concatenate([src, pad_idx]).reshape(E_ROWS_PAD, 128)
    dst2d = jnp.concatenate([dst, pad_idx]).reshape(E_ROWS_PAD, 128)

    npad = N_PAD - N
    xp = jnp.pad(x.astype(_f32), ((0, npad), (0, 0)))
    t0 = _tcpre(xp, W0)
    P0, P1 = _edge_pass_w16e(t0, t0, src2d, dst2d)
    h1 = _tc0(P0, P1, t0)
    P0, P1 = _edge_pass_w16e(h1, h1, src2d, dst2d)
    h2A, h2B = _tc5(P0, P1, h1, W5)
    PA, PB = _edge_pass_w16f(h2A, h2B, src2d, dst2d)
    h3A, h3B = _tc6(PA, PB, W6)
    PA, PB = _edge_pass_w16f(h3A, h3B, src2d, dst2d)

    pxp = jnp.pad(pos[:, 0:1], ((0, npad), (0, 0)))
    pyp = jnp.pad(pos[:, 1:2], ((0, npad), (0, 0)))
    btp = jnp.pad(batch.astype(_i32), (0, npad),
                  constant_values=NB).reshape(N_PAD, 1)
    h4A, h4B, segp = _tc7(PA, PB, W7, h2A, h2B, pxp, pyp, btp)

    parts = _pool(h4A, h4B, segp.reshape(N_PAD))
    pooled = _tcred(parts)
    feat = pooled.reshape(NB, NCELL * 32)
    return _tcfc(feat, Wfc)


# speedup vs baseline: 20.8716x; 1.8004x over previous
import functools

import jax
import jax.numpy as jnp
from jax import lax
from jax.experimental import pallas as pl
from jax.experimental.pallas import tpu as pltpu
import jax.experimental.pallas.tpu_sc as plsc

N = 100000
E = 1600000
NB = 16
GX, GY = 16, 12
NCELL = GX * GY

NC, NS = 2, 16
NW = NC * NS
N_PAD = 100352
E_ROWS = E // 128
E_ROWS_PAD = 12544
RW_INIT = N_PAD // NS
SEG_ROWS = 3264

_f32 = jnp.float32
_i32 = jnp.int32

_SC_MESH = dict(core_axis_name="c", subcore_axis_name="s",
                num_cores=NC, num_subcores=NS)


def _elu(t):
    return jnp.where(t > 0, t, jnp.exp(t) - 1.0)


def _make_edge_pass(w, edge_split):
    rows_w = E_ROWS_PAD // (NW if edge_split else NS)
    GJ = 4
    NG = rows_w // GJ
    H = NG // 2
    assert NG % 2 == 0 and NG * GJ == rows_w
    mesh = plsc.VectorSubcoreMesh(**_SC_MESH)

    def body(tabA, tabB, src2d, dst2d, out0, out1, acc,
             sb0, db0, vb0, sb1, db1, vb1, gsem0, gsem1, ssem0, ssem1):
        c = lax.axis_index("c")
        sid = lax.axis_index("s")
        wid = sid * NC + c
        r0 = sid * RW_INIT

        def run(tab, out):
            pltpu.sync_copy(tab.at[pl.ds(r0, RW_INIT)],
                            acc.at[pl.ds(r0, RW_INIT)])
            plsc.subcore_barrier()
            base0 = (wid if edge_split else sid) * rows_w

            def load_idx(sb, db, grp):
                base = base0 + grp * GJ
                pltpu.sync_copy(src2d.at[pl.ds(base, GJ)], sb)
                pltpu.sync_copy(dst2d.at[pl.ds(base, GJ)], db)

            def fire_gathers(sb, vb, sem):
                for j in range(GJ):
                    pltpu.async_copy(tab.at[sb.at[j]], vb.at[j], sem)

            def wait_gathers(sb, vb, sem):
                for j in range(GJ):
                    pltpu.make_async_copy(tab.at[sb.at[j]], vb.at[j],
                                          sem).wait()

            def scatters(db, vb, sem):
                ds = [pltpu.async_copy(vb.at[j], acc.at[db.at[j]], sem,
                                       add=True) for j in range(GJ)]
                for d in ds:
                    d.wait()

            load_idx(sb0, db0, 0)
            fire_gathers(sb0, vb0, gsem0)

            def pair(t, carry):
                g = 2 * t
                load_idx(sb1, db1, g + 1)
                fire_gathers(sb1, vb1, gsem1)
                wait_gathers(sb0, vb0, gsem0)
                scatters(db0, vb0, ssem0)
                @pl.when(t + 1 < H)
                def _():
                    load_idx(sb0, db0, g + 2)
                    fire_gathers(sb0, vb0, gsem0)
                wait_gathers(sb1, vb1, gsem1)
                scatters(db1, vb1, ssem1)
                return carry

            lax.fori_loop(0, H, pair, 0)
            plsc.subcore_barrier()
            pltpu.sync_copy(acc.at[pl.ds(r0, RW_INIT)],
                            out.at[pl.ds(r0, RW_INIT)])

        @pl.when(c == 0)
        def _():
            run(tabA, out0)

        @pl.when(c == 1)
        def _():
            run(tabB, out1)

    sds = jax.ShapeDtypeStruct((N_PAD, w), _f32)
    idx_t = pltpu.VMEM((GJ, 128), _i32)
    val_t = pltpu.VMEM((GJ, 128, w), _f32)
    return pl.kernel(
        body,
        out_type=[sds, sds],
        mesh=mesh,
        compiler_params=pltpu.CompilerParams(use_tc_tiling_on_sc=False),
        scratch_types=[
            pltpu.VMEM_SHARED((N_PAD, w), _f32),
            idx_t, idx_t, val_t,
            idx_t, idx_t, val_t,
            pltpu.SemaphoreType.DMA,
            pltpu.SemaphoreType.DMA,
            pltpu.SemaphoreType.DMA,
            pltpu.SemaphoreType.DMA,
        ],
    )


_edge_pass_w16e = _make_edge_pass(16, True)
_edge_pass_w16f = _make_edge_pass(16, False)

_POOL_CH = 224
_POOL_NCH = 3136 // _POOL_CH


def _pool_body(hA, hB, px, py, bt, out, acc, bufA, bufB, pxb, pyb, btb):
    c = lax.axis_index("c")
    sid = lax.axis_index("s")
    wid = sid * NC + c
    base = wid * 3136
    neg = jnp.full((16,), -jnp.inf, _f32)

    def ib(r, carry):
        acc[r, pl.ds(0, 16)] = neg
        acc[r, pl.ds(16, 16)] = neg
        return carry

    lax.fori_loop(0, SEG_ROWS, ib, 0)

    def cb(k, carry):
        b = base + k * _POOL_CH
        pltpu.sync_copy(hA.at[pl.ds(b, _POOL_CH)], bufA)
        pltpu.sync_copy(hB.at[pl.ds(b, _POOL_CH)], bufB)
        pltpu.sync_copy(px.at[pl.ds(b, _POOL_CH)], pxb)
        pltpu.sync_copy(py.at[pl.ds(b, _POOL_CH)], pyb)
        pltpu.sync_copy(bt.at[pl.ds(b, _POOL_CH)], btb)

        def nb(g, carry2):
            sl = pl.ds(g * 16, 16)
            cx = jnp.clip((pxb[sl] * GX).astype(_i32), 0, GX - 1)
            cy = jnp.clip((pyb[sl] * GY).astype(_i32), 0, GY - 1)
            segv = btb[sl] * NCELL + cx * GY + cy
            for j in range(16):
                s = segv[j]
                i = g * 16 + j
                acc[s, pl.ds(0, 16)] = jnp.maximum(acc[s, pl.ds(0, 16)],
                                                   bufA[i])
                acc[s, pl.ds(16, 16)] = jnp.maximum(acc[s, pl.ds(16, 16)],
                                                    bufB[i])
            return carry2

        lax.fori_loop(0, _POOL_CH // 16, nb, 0)
        return carry

    lax.fori_loop(0, _POOL_NCH, cb, 0)
    pltpu.sync_copy(acc, out.at[wid])


_pool = pl.kernel(
    _pool_body,
    out_type=jax.ShapeDtypeStruct((NW, SEG_ROWS, 32), _f32),
    mesh=plsc.VectorSubcoreMesh(**_SC_MESH),
    compiler_params=pltpu.CompilerParams(use_tc_tiling_on_sc=False),
    scratch_types=[
        pltpu.VMEM((SEG_ROWS, 32), _f32),
        pltpu.VMEM((_POOL_CH, 16), _f32),
        pltpu.VMEM((_POOL_CH, 16), _f32),
        pltpu.VMEM((_POOL_CH,), _f32),
        pltpu.VMEM((_POOL_CH,), _f32),
        pltpu.VMEM((_POOL_CH,), _i32),
    ],
)


M16 = N_PAD * 16 // 128
_BR = 1568
_NBLK = M16 // _BR
_blk = pl.BlockSpec((_BR, 128), lambda i: (i, 0))
_wblk = pl.BlockSpec((128, 128), lambda i: (0, 0))
_sds2d = jax.ShapeDtypeStruct((M16, 128), _f32)


def _tc0(p0, p1, x16, w0t):
    def body(a_r, b_r, x_r, w_r, o_r):
        o_r[...] = _elu((a_r[...] + b_r[...] - x_r[...]) * w_r[...])

    return pl.pallas_call(
        body,
        grid=(_NBLK,),
        in_specs=[_blk, _blk, _blk, pl.BlockSpec((1, 128), lambda i: (0, 0))],
        out_specs=_blk,
        out_shape=_sds2d,
    )(p0, p1, x16, w0t)


def _tc5(p0, p1, h1, wA, wB):
    def body(a_r, b_r, h_r, wA_r, wB_r, oA, oB):
        t = a_r[...] + b_r[...] - h_r[...]
        oA[...] = _elu(jnp.dot(t, wA_r[...], preferred_element_type=_f32))
        oB[...] = _elu(jnp.dot(t, wB_r[...], preferred_element_type=_f32))

    return pl.pallas_call(
        body,
        grid=(_NBLK,),
        in_specs=[_blk, _blk, _blk, _wblk, _wblk],
        out_specs=[_blk, _blk],
        out_shape=[_sds2d, _sds2d],
    )(p0, p1, h1, wA, wB)


def _tc67(pA, pB, wAA, wBA, wAB, wBB, resA=None, resB=None):
    with_res = resA is not None

    def body(a_r, b_r, wAA_r, wBA_r, wAB_r, wBB_r, *rest):
        if with_res:
            rA_r, rB_r, oA, oB = rest
        else:
            oA, oB = rest
        a = a_r[...]
        b = b_r[...]
        yA = _elu(jnp.dot(a, wAA_r[...], preferred_element_type=_f32)
                  + jnp.dot(b, wBA_r[...], preferred_element_type=_f32))
        yB = _elu(jnp.dot(a, wAB_r[...], preferred_element_type=_f32)
                  + jnp.dot(b, wBB_r[...], preferred_element_type=_f32))
        if with_res:
            yA = yA + rA_r[...]
            yB = yB + rB_r[...]
        oA[...] = yA
        oB[...] = yB

    in_specs = [_blk, _blk, _wblk, _wblk, _wblk, _wblk]
    args = [pA, pB, wAA, wBA, wAB, wBB]
    if with_res:
        in_specs += [_blk, _blk]
        args += [resA, resB]
    return pl.pallas_call(
        body,
        grid=(_NBLK,),
        in_specs=in_specs,
        out_specs=[_blk, _blk],
        out_shape=[_sds2d, _sds2d],
    )(*args)


def _tcred(parts):
    def body(p_r, o_r):
        m = p_r[0]
        for i in range(1, NW):
            m = jnp.maximum(m, p_r[i])
        o_r[...] = jnp.where(jnp.isfinite(m), m, 0.0)

    return pl.pallas_call(
        body,
        grid=(24,),
        in_specs=[pl.BlockSpec((NW, 128, 32), lambda i: (0, i, 0))],
        out_specs=pl.BlockSpec((128, 32), lambda i: (i, 0)),
        out_shape=jax.ShapeDtypeStruct((3072, 32), _f32),
    )(parts)


def _tcfc(feat, Wfc):
    def body(f_r, w_r, o_r):
        o_r[...] = jnp.dot(f_r[...], w_r[...], preferred_element_type=_f32)

    return pl.pallas_call(
        body,
        out_shape=jax.ShapeDtypeStruct((NB, 2), _f32),
    )(feat, Wfc)


def kernel(x, pos, edge_index, batch, W0, W5, W6, W7, Wfc):
    src = edge_index[0].astype(_i32)
    dst = edge_index[1].astype(_i32)
    ep = E_ROWS_PAD * 128 - E
    pad_ar = jnp.arange(ep, dtype=_i32)
    pad_idx = N + pad_ar % (N_PAD - N)
    src2d = jnp.concatenate([src, pad_idx]).reshape(E_ROWS_PAD, 128)
    dst2d = jnp.concatenate([dst, pad_idx]).reshape(E_ROWS_PAD, 128)

    npad = N_PAD - N
    eye8 = jnp.eye(8, dtype=_f32)
    w0t = jnp.tile(W0.reshape(1, 16), (1, 8))
    w5A = jnp.kron(eye8, W5[:, :16])
    w5B = jnp.kron(eye8, W5[:, 16:])
    w6 = [jnp.kron(eye8, W6[r, c]) for r in (slice(0, 16), slice(16, 32))
          for c in (slice(0, 16), slice(16, 32))]
    w7 = [jnp.kron(eye8, W7[r, c]) for r in (slice(0, 16), slice(16, 32))
          for c in (slice(0, 16), slice(16, 32))]

    xb = jnp.pad(x.astype(_f32), ((0, npad), (0, 0)))
    x16 = jnp.broadcast_to(xb, (N_PAD, 16))

    def v2d(a):
        return a.reshape(M16, 128)

    def vsc(a):
        return a.reshape(N_PAD, 16)

    P0, P1 = _edge_pass_w16e(x16, x16, src2d, dst2d)
    h1 = _tc0(v2d(P0), v2d(P1), v2d(x16), w0t)
    P0, P1 = _edge_pass_w16e(vsc(h1), vsc(h1), src2d, dst2d)
    h2A, h2B = _tc5(v2d(P0), v2d(P1), h1, w5A, w5B)
    PA, PB = _edge_pass_w16f(vsc(h2A), vsc(h2B), src2d, dst2d)
    h3A, h3B = _tc67(v2d(PA), v2d(PB), w6[0], w6[2], w6[1], w6[3])
    PA, PB = _edge_pass_w16f(vsc(h3A), vsc(h3B), src2d, dst2d)
    h4A, h4B = _tc67(v2d(PA), v2d(PB), w7[0], w7[2], w7[1], w7[3],
                     resA=h2A, resB=h2B)

    pxp = jnp.pad(pos[:, 0], (0, npad))
    pyp = jnp.pad(pos[:, 1], (0, npad))
    btp = jnp.pad(batch.astype(_i32), (0, npad), constant_values=NB)
    parts = _pool(vsc(h4A), vsc(h4B), pxp, pyp, btp)
    pooled = _tcred(parts)
    feat = pooled.reshape(NB, NCELL * 32)
    return _tcfc(feat, Wfc)


# speedup vs baseline: 25.7482x; 1.2336x over previous
import functools

import jax
import jax.numpy as jnp
import numpy as np
from jax import lax
from jax.experimental import pallas as pl
from jax.experimental.pallas import tpu as pltpu
import jax.experimental.pallas.tpu_sc as plsc

N = 100000
E = 1600000
NB = 16
GX, GY = 16, 12
NCELL = GX * GY

NC, NS = 2, 16
NW = NC * NS
N_PAD = 100352
E_ROWS = E // 128
E_ROWS_PAD = 12544
RW_INIT = N_PAD // NS
SEG_ROWS = 3264

_f32 = jnp.float32
_i32 = jnp.int32

_SC_MESH = dict(core_axis_name="c", subcore_axis_name="s",
                num_cores=NC, num_subcores=NS)

_PAD_SD = np.repeat(
    (np.arange((E_ROWS_PAD - E_ROWS) * 128, dtype=np.int32)
     % (N_PAD - N) + N).reshape(E_ROWS_PAD - E_ROWS, 128),
    2, axis=0)


def _elu(t):
    return jnp.where(t > 0, t, jnp.exp(t) - 1.0)


def _make_edge_pass(w, edge_split):
    rows_w = E_ROWS_PAD // (NW if edge_split else NS)
    GJ = 4
    NG = rows_w // GJ
    H = NG // 2
    assert NG % 2 == 0 and NG * GJ == rows_w
    mesh = plsc.VectorSubcoreMesh(**_SC_MESH)

    def body(tabA, tabB, sd2d, out0, out1, acc,
             sd0, vb0, sd1, vb1, gsem0, gsem1, ssem0, ssem1):
        c = lax.axis_index("c")
        sid = lax.axis_index("s")
        wid = sid * NC + c
        r0 = sid * RW_INIT

        def run(tab, out):
            pltpu.sync_copy(tab.at[pl.ds(r0, RW_INIT)],
                            acc.at[pl.ds(r0, RW_INIT)])
            plsc.subcore_barrier()
            base0 = (wid if edge_split else sid) * rows_w

            def load_idx(sd, grp):
                base = base0 + grp * GJ
                pltpu.sync_copy(sd2d.at[pl.ds(2 * base, 2 * GJ)], sd)

            def fire_gathers(sd, vb, sem):
                for j in range(GJ):
                    pltpu.async_copy(tab.at[sd.at[2 * j]], vb.at[j], sem)

            def wait_gathers(sd, vb, sem):
                for j in range(GJ):
                    pltpu.make_async_copy(tab.at[sd.at[2 * j]], vb.at[j],
                                          sem).wait()

            def scatters(sd, vb, sem):
                ds = [pltpu.async_copy(vb.at[j], acc.at[sd.at[2 * j + 1]],
                                       sem, add=True) for j in range(GJ)]
                for d in ds:
                    d.wait()

            load_idx(sd0, 0)
            fire_gathers(sd0, vb0, gsem0)

            def pair(t, carry):
                g = 2 * t
                load_idx(sd1, g + 1)
                fire_gathers(sd1, vb1, gsem1)
                wait_gathers(sd0, vb0, gsem0)
                scatters(sd0, vb0, ssem0)
                @pl.when(t + 1 < H)
                def _():
                    load_idx(sd0, g + 2)
                    fire_gathers(sd0, vb0, gsem0)
                wait_gathers(sd1, vb1, gsem1)
                scatters(sd1, vb1, ssem1)
                return carry

            lax.fori_loop(0, H, pair, 0)
            plsc.subcore_barrier()
            pltpu.sync_copy(acc.at[pl.ds(r0, RW_INIT)],
                            out.at[pl.ds(r0, RW_INIT)])

        @pl.when(c == 0)
        def _():
            run(tabA, out0)

        @pl.when(c == 1)
        def _():
            run(tabB, out1)

    sds = jax.ShapeDtypeStruct((N_PAD, w), _f32)
    idx_t = pltpu.VMEM((2 * GJ, 128), _i32)
    val_t = pltpu.VMEM((GJ, 128, w), _f32)
    return pl.kernel(
        body,
        out_type=[sds, sds],
        mesh=mesh,
        compiler_params=pltpu.CompilerParams(use_tc_tiling_on_sc=False),
        scratch_types=[
            pltpu.VMEM_SHARED((N_PAD, w), _f32),
            idx_t, val_t,
            idx_t, val_t,
            pltpu.SemaphoreType.DMA,
            pltpu.SemaphoreType.DMA,
            pltpu.SemaphoreType.DMA,
            pltpu.SemaphoreType.DMA,
        ],
    )


_edge_pass_w16e = _make_edge_pass(16, True)
_edge_pass_w16f = _make_edge_pass(16, False)

_POOL_CH = 224
_POOL_NCH = 3136 // _POOL_CH


def _pool_body(hA, hB, px, py, bt, out, acc, bufA, bufB, pxb, pyb, btb):
    c = lax.axis_index("c")
    sid = lax.axis_index("s")
    wid = sid * NC + c
    base = wid * 3136
    neg = jnp.full((16,), -jnp.inf, _f32)

    def ib(r, carry):
        acc[r, pl.ds(0, 16)] = neg
        acc[r, pl.ds(16, 16)] = neg
        return carry

    lax.fori_loop(0, SEG_ROWS, ib, 0)

    def cb(k, carry):
        b = base + k * _POOL_CH
        pltpu.sync_copy(hA.at[pl.ds(b, _POOL_CH)], bufA)
        pltpu.sync_copy(hB.at[pl.ds(b, _POOL_CH)], bufB)
        pltpu.sync_copy(px.at[pl.ds(b, _POOL_CH)], pxb)
        pltpu.sync_copy(py.at[pl.ds(b, _POOL_CH)], pyb)
        pltpu.sync_copy(bt.at[pl.ds(b, _POOL_CH)], btb)

        def nb(g, carry2):
            sl = pl.ds(g * 16, 16)
            cx = jnp.clip((pxb[sl] * GX).astype(_i32), 0, GX - 1)
            cy = jnp.clip((pyb[sl] * GY).astype(_i32), 0, GY - 1)
            segv = btb[sl] * NCELL + cx * GY + cy
            for j in range(16):
                s = segv[j]
                i = g * 16 + j
                acc[s, pl.ds(0, 16)] = jnp.maximum(acc[s, pl.ds(0, 16)],
                                                   bufA[i])
                acc[s, pl.ds(16, 16)] = jnp.maximum(acc[s, pl.ds(16, 16)],
                                                    bufB[i])
            return carry2

        lax.fori_loop(0, _POOL_CH // 16, nb, 0)
        return carry

    lax.fori_loop(0, _POOL_NCH, cb, 0)
    pltpu.sync_copy(acc, out.at[wid])


_pool = pl.kernel(
    _pool_body,
    out_type=jax.ShapeDtypeStruct((NW, SEG_ROWS, 32), _f32),
    mesh=plsc.VectorSubcoreMesh(**_SC_MESH),
    compiler_params=pltpu.CompilerParams(use_tc_tiling_on_sc=False),
    scratch_types=[
        pltpu.VMEM((SEG_ROWS, 32), _f32),
        pltpu.VMEM((_POOL_CH, 16), _f32),
        pltpu.VMEM((_POOL_CH, 16), _f32),
        pltpu.VMEM((_POOL_CH,), _f32),
        pltpu.VMEM((_POOL_CH,), _f32),
        pltpu.VMEM((_POOL_CH,), _i32),
    ],
)


M16 = N_PAD * 16 // 128
_BR = 1568
_NBLK = M16 // _BR
_blk = pl.BlockSpec((_BR, 128), lambda i: (i, 0))
_wblk = pl.BlockSpec((128, 128), lambda i: (0, 0))
_sds2d = jax.ShapeDtypeStruct((M16, 128), _f32)


def _tc0(p0, p1, x16, w0t):
    def body(a_r, b_r, x_r, w_r, o_r):
        o_r[...] = _elu((a_r[...] + b_r[...] - x_r[...]) * w_r[...])

    return pl.pallas_call(
        body,
        grid=(_NBLK,),
        in_specs=[_blk, _blk, _blk, pl.BlockSpec((1, 128), lambda i: (0, 0))],
        out_specs=_blk,
        out_shape=_sds2d,
    )(p0, p1, x16, w0t)


def _tc5(p0, p1, h1, wA, wB):
    def body(a_r, b_r, h_r, wA_r, wB_r, oA, oB):
        t = a_r[...] + b_r[...] - h_r[...]
        oA[...] = _elu(jnp.dot(t, wA_r[...], preferred_element_type=_f32))
        oB[...] = _elu(jnp.dot(t, wB_r[...], preferred_element_type=_f32))

    return pl.pallas_call(
        body,
        grid=(_NBLK,),
        in_specs=[_blk, _blk, _blk, _wblk, _wblk],
        out_specs=[_blk, _blk],
        out_shape=[_sds2d, _sds2d],
    )(p0, p1, h1, wA, wB)


def _tc67(pA, pB, wAA, wBA, wAB, wBB, resA=None, resB=None):
    with_res = resA is not None

    def body(a_r, b_r, wAA_r, wBA_r, wAB_r, wBB_r, *rest):
        if with_res:
            rA_r, rB_r, oA, oB = rest
        else:
            oA, oB = rest
        a = a_r[...]
        b = b_r[...]
        yA = _elu(jnp.dot(a, wAA_r[...], preferred_element_type=_f32)
                  + jnp.dot(b, wBA_r[...], preferred_element_type=_f32))
        yB = _elu(jnp.dot(a, wAB_r[...], preferred_element_type=_f32)
                  + jnp.dot(b, wBB_r[...], preferred_element_type=_f32))
        if with_res:
            yA = yA + rA_r[...]
            yB = yB + rB_r[...]
        oA[...] = yA
        oB[...] = yB

    in_specs = [_blk, _blk, _wblk, _wblk, _wblk, _wblk]
    args = [pA, pB, wAA, wBA, wAB, wBB]
    if with_res:
        in_specs += [_blk, _blk]
        args += [resA, resB]
    return pl.pallas_call(
        body,
        grid=(_NBLK,),
        in_specs=in_specs,
        out_specs=[_blk, _blk],
        out_shape=[_sds2d, _sds2d],
    )(*args)


def _tcred(partsv):
    def body(p_r, o_r):
        m = p_r[0]
        for i in range(1, NW):
            m = jnp.maximum(m, p_r[i])
        o_r[...] = jnp.where(jnp.isfinite(m), m, 0.0)

    return pl.pallas_call(
        body,
        grid=(6,),
        in_specs=[pl.BlockSpec((NW, 128, 128), lambda i: (0, i, 0))],
        out_specs=pl.BlockSpec((128, 128), lambda i: (i, 0)),
        out_shape=jax.ShapeDtypeStruct((768, 128), _f32),
    )(partsv)


def _tcfc(feat, Wfc):
    def body(f_r, w_r, o_r):
        o_r[...] = jnp.dot(f_r[...], w_r[...], preferred_element_type=_f32)

    return pl.pallas_call(
        body,
        out_shape=jax.ShapeDtypeStruct((NB, 2), _f32),
    )(feat, Wfc)


def kernel(x, pos, edge_index, batch, W0, W5, W6, W7, Wfc):
    src = edge_index[0].astype(_i32)
    dst = edge_index[1].astype(_i32)
    sd_main = jnp.stack([src.reshape(E_ROWS, 128),
                         dst.reshape(E_ROWS, 128)], axis=1)
    sd_main = sd_main.reshape(2 * E_ROWS, 128)
    sd2d = jnp.concatenate([sd_main, jnp.asarray(_PAD_SD)], axis=0)

    npad = N_PAD - N
    eye8 = jnp.eye(8, dtype=_f32)
    w0t = jnp.tile(W0.reshape(1, 16), (1, 8))
    w5A = jnp.kron(eye8, W5[:, :16])
    w5B = jnp.kron(eye8, W5[:, 16:])
    w6 = [jnp.kron(eye8, W6[r, c]) for r in (slice(0, 16), slice(16, 32))
          for c in (slice(0, 16), slice(16, 32))]
    w7 = [jnp.kron(eye8, W7[r, c]) for r in (slice(0, 16), slice(16, 32))
          for c in (slice(0, 16), slice(16, 32))]

    xpad = jnp.pad(x.astype(_f32).reshape(N), (0, npad))
    x16p = jnp.repeat(xpad.reshape(M16, 8), 16, axis=1)
    x16 = x16p.reshape(N_PAD, 16)

    def v2d(a):
        return a.reshape(M16, 128)

    def vsc(a):
        return a.reshape(N_PAD, 16)

    P0, P1 = _edge_pass_w16e(x16, x16, sd2d)
    h1 = _tc0(v2d(P0), v2d(P1), x16p, w0t)
    P0, P1 = _edge_pass_w16e(vsc(h1), vsc(h1), sd2d)
    h2A, h2B = _tc5(v2d(P0), v2d(P1), h1, w5A, w5B)
    PA, PB = _edge_pass_w16f(vsc(h2A), vsc(h2B), sd2d)
    h3A, h3B = _tc67(v2d(PA), v2d(PB), w6[0], w6[2], w6[1], w6[3])
    PA, PB = _edge_pass_w16f(vsc(h3A), vsc(h3B), sd2d)
    h4A, h4B = _tc67(v2d(PA), v2d(PB), w7[0], w7[2], w7[1], w7[3],
                     resA=h2A, resB=h2B)

    pxp = jnp.pad(pos[:, 0], (0, npad))
    pyp = jnp.pad(pos[:, 1], (0, npad))
    btp = jnp.pad(batch.astype(_i32), (0, npad), constant_values=NB)
    parts = _pool(vsc(h4A), vsc(h4B), pxp, pyp, btp)
    pooled = _tcred(parts.reshape(NW, 816, 128))
    feat = pooled.reshape(NB, NCELL * 32)
    return _tcfc(feat, Wfc)


# speedup vs baseline: 25.7629x; 1.0006x over previous
import functools

import jax
import jax.numpy as jnp
import numpy as np
from jax import lax
from jax.experimental import pallas as pl
from jax.experimental.pallas import tpu as pltpu
import jax.experimental.pallas.tpu_sc as plsc

N = 100000
E = 1600000
NB = 16
GX, GY = 16, 12
NCELL = GX * GY

NC, NS = 2, 16
NW = NC * NS
N_PAD = 100352
E_ROWS = E // 128
E_ROWS_PAD = 12544
RW_INIT = N_PAD // NS
SEG_ROWS = 3264

_f32 = jnp.float32
_i32 = jnp.int32

_SC_MESH = dict(core_axis_name="c", subcore_axis_name="s",
                num_cores=NC, num_subcores=NS)

_PAD_ROWS = (np.arange((E_ROWS_PAD - E_ROWS) * 128, dtype=np.int32)
             % (N_PAD - N) + N).reshape(E_ROWS_PAD - E_ROWS, 128)


def _elu(t):
    return jnp.where(t > 0, t, jnp.exp(t) - 1.0)


def _make_edge_pass(w, edge_split):
    rows_w = E_ROWS_PAD // (NW if edge_split else NS)
    GJ = 4
    NG = rows_w // GJ
    H = NG // 2
    assert NG % 2 == 0 and NG * GJ == rows_w
    mesh = plsc.VectorSubcoreMesh(**_SC_MESH)

    def body(tabA, tabB, sd2d, out0, out1, acc,
             sd0, vb0, sd1, vb1, gsem0, gsem1, ssem0, ssem1):
        c = lax.axis_index("c")
        sid = lax.axis_index("s")
        wid = sid * NC + c
        r0 = sid * RW_INIT

        def run(tab, out):
            pltpu.sync_copy(tab.at[pl.ds(r0, RW_INIT)],
                            acc.at[pl.ds(r0, RW_INIT)])
            plsc.subcore_barrier()
            base0 = (wid if edge_split else sid) * rows_w

            g0 = base0 // GJ

            def load_idx(sd, grp):
                pltpu.sync_copy(sd2d.at[pl.ds((g0 + grp) * 2, 2)], sd)

            def fire_gathers(sd, vb, sem):
                pltpu.async_copy(tab.at[sd.at[0]], vb, sem)

            def wait_gathers(sd, vb, sem):
                pltpu.make_async_copy(tab.at[sd.at[0]], vb, sem).wait()

            def scatters(sd, vb, sem):
                pltpu.async_copy(vb, acc.at[sd.at[1]], sem, add=True).wait()

            load_idx(sd0, 0)
            fire_gathers(sd0, vb0, gsem0)

            def pair(t, carry):
                g = 2 * t
                load_idx(sd1, g + 1)
                fire_gathers(sd1, vb1, gsem1)
                wait_gathers(sd0, vb0, gsem0)
                scatters(sd0, vb0, ssem0)
                @pl.when(t + 1 < H)
                def _():
                    load_idx(sd0, g + 2)
                    fire_gathers(sd0, vb0, gsem0)
                wait_gathers(sd1, vb1, gsem1)
                scatters(sd1, vb1, ssem1)
                return carry

            lax.fori_loop(0, H, pair, 0)
            plsc.subcore_barrier()
            pltpu.sync_copy(acc.at[pl.ds(r0, RW_INIT)],
                            out.at[pl.ds(r0, RW_INIT)])

        @pl.when(c == 0)
        def _():
            run(tabA, out0)

        @pl.when(c == 1)
        def _():
            run(tabB, out1)

    sds = jax.ShapeDtypeStruct((N_PAD, w), _f32)
    idx_t = pltpu.VMEM((2, GJ * 128), _i32)
    val_t = pltpu.VMEM((GJ * 128, w), _f32)
    return pl.kernel(
        body,
        out_type=[sds, sds],
        mesh=mesh,
        compiler_params=pltpu.CompilerParams(use_tc_tiling_on_sc=False),
        scratch_types=[
            pltpu.VMEM_SHARED((N_PAD, w), _f32),
            idx_t, val_t,
            idx_t, val_t,
            pltpu.SemaphoreType.DMA,
            pltpu.SemaphoreType.DMA,
            pltpu.SemaphoreType.DMA,
            pltpu.SemaphoreType.DMA,
        ],
    )


_edge_pass_w16e = _make_edge_pass(16, True)
_edge_pass_w16f = _make_edge_pass(16, False)

_POOL_CH = 224
_POOL_NCH = 3136 // _POOL_CH


def _pool_body(hA, hB, px, py, bt, out, acc, bufA, bufB, pxb, pyb, btb):
    c = lax.axis_index("c")
    sid = lax.axis_index("s")
    wid = sid * NC + c
    base = wid * 3136
    neg = jnp.full((16,), -jnp.inf, _f32)

    def ib(r, carry):
        acc[r, pl.ds(0, 16)] = neg
        acc[r, pl.ds(16, 16)] = neg
        return carry

    lax.fori_loop(0, SEG_ROWS, ib, 0)

    def cb(k, carry):
        b = base + k * _POOL_CH
        pltpu.sync_copy(hA.at[pl.ds(b, _POOL_CH)], bufA)
        pltpu.sync_copy(hB.at[pl.ds(b, _POOL_CH)], bufB)
        pltpu.sync_copy(px.at[pl.ds(b, _POOL_CH)], pxb)
        pltpu.sync_copy(py.at[pl.ds(b, _POOL_CH)], pyb)
        pltpu.sync_copy(bt.at[pl.ds(b, _POOL_CH)], btb)

        def nb(g, carry2):
            sl = pl.ds(g * 16, 16)
            cx = jnp.clip((pxb[sl] * GX).astype(_i32), 0, GX - 1)
            cy = jnp.clip((pyb[sl] * GY).astype(_i32), 0, GY - 1)
            segv = btb[sl] * NCELL + cx * GY + cy
            for j in range(16):
                s = segv[j]
                i = g * 16 + j
                acc[s, pl.ds(0, 16)] = jnp.maximum(acc[s, pl.ds(0, 16)],
                                                   bufA[i])
                acc[s, pl.ds(16, 16)] = jnp.maximum(acc[s, pl.ds(16, 16)],
                                                    bufB[i])
            return carry2

        lax.fori_loop(0, _POOL_CH // 16, nb, 0)
        return carry

    lax.fori_loop(0, _POOL_NCH, cb, 0)
    pltpu.sync_copy(acc, out.at[wid])


_pool = pl.kernel(
    _pool_body,
    out_type=jax.ShapeDtypeStruct((NW, SEG_ROWS, 32), _f32),
    mesh=plsc.VectorSubcoreMesh(**_SC_MESH),
    compiler_params=pltpu.CompilerParams(use_tc_tiling_on_sc=False),
    scratch_types=[
        pltpu.VMEM((SEG_ROWS, 32), _f32),
        pltpu.VMEM((_POOL_CH, 16), _f32),
        pltpu.VMEM((_POOL_CH, 16), _f32),
        pltpu.VMEM((_POOL_CH,), _f32),
        pltpu.VMEM((_POOL_CH,), _f32),
        pltpu.VMEM((_POOL_CH,), _i32),
    ],
)


M16 = N_PAD * 16 // 128
_BR = 1568
_NBLK = M16 // _BR
_blk = pl.BlockSpec((_BR, 128), lambda i: (i, 0))
_wblk = pl.BlockSpec((128, 128), lambda i: (0, 0))
_sds2d = jax.ShapeDtypeStruct((M16, 128), _f32)


def _tc0(p0, p1, x16, w0t):
    def body(a_r, b_r, x_r, w_r, o_r):
        o_r[...] = _elu((a_r[...] + b_r[...] - x_r[...]) * w_r[...])

    return pl.pallas_call(
        body,
        grid=(_NBLK,),
        in_specs=[_blk, _blk, _blk, pl.BlockSpec((1, 128), lambda i: (0, 0))],
        out_specs=_blk,
        out_shape=_sds2d,
    )(p0, p1, x16, w0t)


def _tc5(p0, p1, h1, wA, wB):
    def body(a_r, b_r, h_r, wA_r, wB_r, oA, oB):
        t = a_r[...] + b_r[...] - h_r[...]
        oA[...] = _elu(jnp.dot(t, wA_r[...], preferred_element_type=_f32))
        oB[...] = _elu(jnp.dot(t, wB_r[...], preferred_element_type=_f32))

    return pl.pallas_call(
        body,
        grid=(_NBLK,),
        in_specs=[_blk, _blk, _blk, _wblk, _wblk],
        out_specs=[_blk, _blk],
        out_shape=[_sds2d, _sds2d],
    )(p0, p1, h1, wA, wB)


def _tc67(pA, pB, wAA, wBA, wAB, wBB, resA=None, resB=None):
    with_res = resA is not None

    def body(a_r, b_r, wAA_r, wBA_r, wAB_r, wBB_r, *rest):
        if with_res:
            rA_r, rB_r, oA, oB = rest
        else:
            oA, oB = rest
        a = a_r[...]
        b = b_r[...]
        yA = _elu(jnp.dot(a, wAA_r[...], preferred_element_type=_f32)
                  + jnp.dot(b, wBA_r[...], preferred_element_type=_f32))
        yB = _elu(jnp.dot(a, wAB_r[...], preferred_element_type=_f32)
                  + jnp.dot(b, wBB_r[...], preferred_element_type=_f32))
        if with_res:
            yA = yA + rA_r[...]
            yB = yB + rB_r[...]
        oA[...] = yA
        oB[...] = yB

    in_specs = [_blk, _blk, _wblk, _wblk, _wblk, _wblk]
    args = [pA, pB, wAA, wBA, wAB, wBB]
    if with_res:
        in_specs += [_blk, _blk]
        args += [resA, resB]
    return pl.pallas_call(
        body,
        grid=(_NBLK,),
        in_specs=in_specs,
        out_specs=[_blk, _blk],
        out_shape=[_sds2d, _sds2d],
    )(*args)


def _tcred(partsv):
    def body(p_r, o_r):
        m = p_r[0]
        for i in range(1, NW):
            m = jnp.maximum(m, p_r[i])
        o_r[...] = jnp.where(jnp.isfinite(m), m, 0.0)

    return pl.pallas_call(
        body,
        grid=(6,),
        in_specs=[pl.BlockSpec((NW, 128, 128), lambda i: (0, i, 0))],
        out_specs=pl.BlockSpec((128, 128), lambda i: (i, 0)),
        out_shape=jax.ShapeDtypeStruct((768, 128), _f32),
    )(partsv)


def _tcfc(feat, Wfc):
    def body(f_r, w_r, o_r):
        o_r[...] = jnp.dot(f_r[...], w_r[...], preferred_element_type=_f32)

    return pl.pallas_call(
        body,
        out_shape=jax.ShapeDtypeStruct((NB, 2), _f32),
    )(feat, Wfc)


def kernel(x, pos, edge_index, batch, W0, W5, W6, W7, Wfc):
    src = edge_index[0].astype(_i32)
    dst = edge_index[1].astype(_i32)
    pad = jnp.asarray(_PAD_ROWS)
    s2 = jnp.concatenate([src.reshape(E_ROWS, 128), pad], axis=0)
    d2 = jnp.concatenate([dst.reshape(E_ROWS, 128), pad], axis=0)
    ngt = E_ROWS_PAD // 4
    sd2d = jnp.stack([s2.reshape(ngt, 512), d2.reshape(ngt, 512)],
                     axis=1).reshape(2 * ngt, 512)

    npad = N_PAD - N
    eye8 = jnp.eye(8, dtype=_f32)
    w0t = jnp.tile(W0.reshape(1, 16), (1, 8))
    w5A = jnp.kron(eye8, W5[:, :16])
    w5B = jnp.kron(eye8, W5[:, 16:])
    w6 = [jnp.kron(eye8, W6[r, c]) for r in (slice(0, 16), slice(16, 32))
          for c in (slice(0, 16), slice(16, 32))]
    w7 = [jnp.kron(eye8, W7[r, c]) for r in (slice(0, 16), slice(16, 32))
          for c in (slice(0, 16), slice(16, 32))]

    xpad = jnp.pad(x.astype(_f32).reshape(N), (0, npad))
    x16p = jnp.repeat(xpad.reshape(M16, 8), 16, axis=1)
    x16 = x16p.reshape(N_PAD, 16)

    def v2d(a):
        return a.reshape(M16, 128)

    def vsc(a):
        return a.reshape(N_PAD, 16)

    P0, P1 = _edge_pass_w16e(x16, x16, sd2d)
    h1 = _tc0(v2d(P0), v2d(P1), x16p, w0t)
    P0, P1 = _edge_pass_w16e(vsc(h1), vsc(h1), sd2d)
    h2A, h2B = _tc5(v2d(P0), v2d(P1), h1, w5A, w5B)
    PA, PB = _edge_pass_w16f(vsc(h2A), vsc(h2B), sd2d)
    h3A, h3B = _tc67(v2d(PA), v2d(PB), w6[0], w6[2], w6[1], w6[3])
    PA, PB = _edge_pass_w16f(vsc(h3A), vsc(h3B), sd2d)
    h4A, h4B = _tc67(v2d(PA), v2d(PB), w7[0], w7[2], w7[1], w7[3],
                     resA=h2A, resB=h2B)

    pxp = jnp.pad(pos[:, 0], (0, npad))
    pyp = jnp.pad(pos[:, 1], (0, npad))
    btp = jnp.pad(batch.astype(_i32), (0, npad), constant_values=NB)
    parts = _pool(vsc(h4A), vsc(h4B), pxp, pyp, btp)
    pooled = _tcred(parts.reshape(NW, 816, 128))
    feat = pooled.reshape(NB, NCELL * 32)
    return _tcfc(feat, Wfc)
